# Initial kernel scaffold; baseline (speedup 1.0000x reference)
import jax
import jax.numpy as jnp
from jax.experimental import pallas as pl


def kernel(x, edges, W, b, gamma, beta):
    raise NotImplementedError("write your pallas kernel here")


# speedup vs baseline: 9.3953x; 9.3953x over previous
import functools

import jax
import jax.numpy as jnp
from jax import lax
from jax.experimental import pallas as pl
from jax.experimental.pallas import tpu as pltpu
from jax.experimental.pallas import tpu_sc as plsc

N = 10000
D = 128
NC = 2
NS = 16
NW = NC * NS
CHUNK = 128
N_PAD = 10240
ROWS_PER_SUB = N_PAD // NS
E_PER_TILE_CHUNKS = None


def _sc_segment_sum(x, src_t, dst_t, nch):
    mesh = plsc.VectorSubcoreMesh(
        core_axis_name="c", subcore_axis_name="s", num_cores=NC,
        num_subcores=NS)

    @functools.partial(
        pl.kernel,
        out_type=(
            jax.ShapeDtypeStruct((NC, N_PAD, D), jnp.float32),
            jax.ShapeDtypeStruct((NC, N_PAD), jnp.float32),
        ),
        mesh=mesh,
        scratch_types=[
            pltpu.VMEM((nch, CHUNK), jnp.int32),
            pltpu.VMEM((nch, CHUNK), jnp.int32),
            pltpu.VMEM((CHUNK, D), jnp.float32),
            pltpu.VMEM((CHUNK,), jnp.float32),
            pltpu.VMEM((CHUNK,), jnp.float32),
            pltpu.VMEM_SHARED((N_PAD, D), jnp.float32),
            pltpu.VMEM_SHARED((N_PAD,), jnp.float32),
            pltpu.SemaphoreType.DMA,
        ],
    )
    def k(x_hbm, src_hbm, dst_hbm, agg_out, deg_out,
          src_v, dst_v, rows_v, ones_v, zed_v, agg_sh, deg_sh, sem):
        c = lax.axis_index("c")
        s = lax.axis_index("s")
        wid = c * NS + s
        row0 = s * ROWS_PER_SUB

        z16 = jnp.zeros((16,), jnp.float32)
        o16 = jnp.ones((16,), jnp.float32)
        for j in range(CHUNK // 16):
            ones_v[pl.ds(j * 16, 16)] = o16
            zed_v[pl.ds(j * 16, 16)] = z16

        def zrow(i, carry):
            for j in range(D // 16):
                rows_v[i, pl.ds(j * 16, 16)] = z16
            return carry
        lax.fori_loop(0, CHUNK, zrow, 0)

        for kk in range(ROWS_PER_SUB // CHUNK):
            pltpu.sync_copy(rows_v, agg_sh.at[pl.ds(row0 + kk * CHUNK, CHUNK)])
            pltpu.sync_copy(zed_v, deg_sh.at[pl.ds(row0 + kk * CHUNK, CHUNK)])
        plsc.subcore_barrier()

        pltpu.sync_copy(src_hbm.at[wid], src_v)
        pltpu.sync_copy(dst_hbm.at[wid], dst_v)

        def body(j, carry):
            pltpu.async_copy(x_hbm.at[src_v.at[j]], rows_v, sem).wait()
            pltpu.sync_copy(rows_v, agg_sh.at[dst_v.at[j]], add=True)
            pltpu.sync_copy(ones_v, deg_sh.at[dst_v.at[j]], add=True)
            return carry
        lax.fori_loop(0, nch, body, 0)

        plsc.subcore_barrier()

        pltpu.sync_copy(agg_sh.at[pl.ds(row0, ROWS_PER_SUB)],
                        agg_out.at[c, pl.ds(row0, ROWS_PER_SUB)])
        pltpu.sync_copy(deg_sh.at[pl.ds(row0, ROWS_PER_SUB)],
                        deg_out.at[c, pl.ds(row0, ROWS_PER_SUB)])

    return k(x, src_t, dst_t)


def _tc_finale(agg0, agg1, d0, d1, x, W, b, gamma, beta):
    def body(a0_ref, a1_ref, d0_ref, d1_ref, x_ref, w_ref, b_ref, g_ref,
             be_ref, o_ref):
        a = a0_ref[...][:N] + a1_ref[...][:N]
        deg = d0_ref[...][:N] + d1_ref[...][:N]
        m = a / jnp.maximum(deg, 1.0)
        o = jnp.dot(m, w_ref[...], preferred_element_type=jnp.float32)
        o = o + b_ref[...]
        mu = jnp.mean(o, axis=0, keepdims=True)
        var = jnp.mean((o - mu) * (o - mu), axis=0, keepdims=True)
        o = (o - mu) * lax.rsqrt(var + 1e-5) * g_ref[...] + be_ref[...]
        o = jnp.where(o > 0.0, o, jnp.exp(jnp.minimum(o, 0.0)) - 1.0)
        o_ref[...] = x_ref[...] + o

    return pl.pallas_call(
        body,
        out_shape=jax.ShapeDtypeStruct((N, D), jnp.float32),
    )(agg0, agg1, d0, d1, x, W, b, gamma, beta)


def kernel(x, edges, W, b, gamma, beta):
    E = edges.shape[1]
    src = edges[0].astype(jnp.int32)
    dst = edges[1].astype(jnp.int32)

    e_per_tile = -(-E // NW)
    nch = -(-e_per_tile // CHUNK)
    e_pad = NW * nch * CHUNK
    npad = e_pad - E
    if npad:
        ar = jnp.arange(npad, dtype=jnp.int32)
        src = jnp.concatenate([src, ar % N])
        dst = jnp.concatenate([dst, N + (ar % (N_PAD - N))])
    src_t = src.reshape(NW, nch, CHUNK)
    dst_t = dst.reshape(NW, nch, CHUNK)

    agg, deg = _sc_segment_sum(x, src_t, dst_t, nch)

    return _tc_finale(
        agg[0], agg[1],
        deg[0].reshape(N_PAD, 1), deg[1].reshape(N_PAD, 1),
        x, W,
        b.reshape(1, D), gamma.reshape(1, D), beta.reshape(1, D),
    )


# speedup vs baseline: 12.6730x; 1.3489x over previous
import functools

import jax
import jax.numpy as jnp
from jax import lax
from jax.experimental import pallas as pl
from jax.experimental.pallas import tpu as pltpu
from jax.experimental.pallas import tpu_sc as plsc

N = 10000
D = 128
NC = 2
NS = 16
NW = NC * NS
CHUNK = 128
WIN = 16
N_PAD = 10240
ROWS_PER_SUB = N_PAD // NS
E_PER_TILE_CHUNKS = None


def _sc_segment_sum(x, src_t, dst_t, nch):
    mesh = plsc.VectorSubcoreMesh(
        core_axis_name="c", subcore_axis_name="s", num_cores=NC,
        num_subcores=NS)

    @functools.partial(
        pl.kernel,
        out_type=(
            jax.ShapeDtypeStruct((NC, N_PAD, D), jnp.float32),
            jax.ShapeDtypeStruct((NC, N_PAD), jnp.float32),
        ),
        mesh=mesh,
        scratch_types=[
            pltpu.VMEM((WIN, CHUNK), jnp.int32),
            pltpu.VMEM((WIN, CHUNK), jnp.int32),
            pltpu.VMEM((CHUNK, D), jnp.float32),
            pltpu.VMEM((CHUNK, D), jnp.float32),
            pltpu.VMEM((CHUNK,), jnp.float32),
            pltpu.VMEM((CHUNK,), jnp.float32),
            pltpu.VMEM_SHARED((N_PAD, D), jnp.float32),
            pltpu.VMEM_SHARED((N_PAD,), jnp.float32),
            pltpu.SemaphoreType.DMA,
            pltpu.SemaphoreType.DMA,
        ],
    )
    def k(x_hbm, src_hbm, dst_hbm, agg_out, deg_out,
          src_v, dst_v, rows0_v, rows1_v, ones_v, zed_v, agg_sh, deg_sh,
          sem0, sem1):
        rows = (rows0_v, rows1_v)
        sems = (sem0, sem1)
        c = lax.axis_index("c")
        s = lax.axis_index("s")
        wid = c * NS + s
        row0 = s * ROWS_PER_SUB

        z16 = jnp.zeros((16,), jnp.float32)
        o16 = jnp.ones((16,), jnp.float32)
        for j in range(CHUNK // 16):
            ones_v[pl.ds(j * 16, 16)] = o16
            zed_v[pl.ds(j * 16, 16)] = z16

        def zrow(i, carry):
            for j in range(D // 16):
                rows0_v[i, pl.ds(j * 16, 16)] = z16
            return carry
        lax.fori_loop(0, CHUNK, zrow, 0)

        for kk in range(ROWS_PER_SUB // CHUNK):
            pltpu.sync_copy(rows0_v, agg_sh.at[pl.ds(row0 + kk * CHUNK, CHUNK)])
            pltpu.sync_copy(zed_v, deg_sh.at[pl.ds(row0 + kk * CHUNK, CHUNK)])
        plsc.subcore_barrier()

        def win_body(w, carry):
            pltpu.sync_copy(src_hbm.at[wid, pl.ds(w * WIN, WIN)], src_v)
            pltpu.sync_copy(dst_hbm.at[wid, pl.ds(w * WIN, WIN)], dst_v)
            pltpu.async_copy(x_hbm.at[src_v.at[0]], rows0_v, sem0)

            def body(p, c):
                for t in range(2):
                    lj = p * 2 + t
                    nxt = lj + 1

                    @pl.when(nxt < WIN)
                    def _():
                        pltpu.async_copy(x_hbm.at[src_v.at[nxt]],
                                         rows[1 - t], sems[1 - t])

                    pltpu.make_async_copy(x_hbm.at[src_v.at[lj]],
                                          rows[t], sems[t]).wait()
                    pltpu.sync_copy(rows[t], agg_sh.at[dst_v.at[lj]],
                                    add=True)
                    pltpu.sync_copy(ones_v, deg_sh.at[dst_v.at[lj]],
                                    add=True)
                return c
            lax.fori_loop(0, WIN // 2, body, 0)
            return carry
        lax.fori_loop(0, nch // WIN, win_body, 0)

        plsc.subcore_barrier()

        pltpu.sync_copy(agg_sh.at[pl.ds(row0, ROWS_PER_SUB)],
                        agg_out.at[c, pl.ds(row0, ROWS_PER_SUB)])
        pltpu.sync_copy(deg_sh.at[pl.ds(row0, ROWS_PER_SUB)],
                        deg_out.at[c, pl.ds(row0, ROWS_PER_SUB)])

    return k(x, src_t, dst_t)


def _tc_finale(agg0, agg1, d0, d1, x, W, b, gamma, beta):
    def body(a0_ref, a1_ref, d0_ref, d1_ref, x_ref, w_ref, b_ref, g_ref,
             be_ref, o_ref):
        a = a0_ref[...][:N] + a1_ref[...][:N]
        deg = d0_ref[...][:N] + d1_ref[...][:N]
        m = a / jnp.maximum(deg, 1.0)
        o = jnp.dot(m, w_ref[...], preferred_element_type=jnp.float32)
        o = o + b_ref[...]
        mu = jnp.mean(o, axis=0, keepdims=True)
        var = jnp.mean((o - mu) * (o - mu), axis=0, keepdims=True)
        o = (o - mu) * lax.rsqrt(var + 1e-5) * g_ref[...] + be_ref[...]
        o = jnp.where(o > 0.0, o, jnp.exp(jnp.minimum(o, 0.0)) - 1.0)
        o_ref[...] = x_ref[...] + o

    return pl.pallas_call(
        body,
        out_shape=jax.ShapeDtypeStruct((N, D), jnp.float32),
    )(agg0, agg1, d0, d1, x, W, b, gamma, beta)


def kernel(x, edges, W, b, gamma, beta):
    E = edges.shape[1]
    src = edges[0].astype(jnp.int32)
    dst = edges[1].astype(jnp.int32)

    e_per_tile = -(-E // NW)
    nch = -(-e_per_tile // CHUNK)
    nch = -(-nch // WIN) * WIN
    e_pad = NW * nch * CHUNK
    npad = e_pad - E
    if npad:
        ar = jnp.arange(npad, dtype=jnp.int32)
        src = jnp.concatenate([src, ar % N])
        dst = jnp.concatenate([dst, N + (ar % (N_PAD - N))])
    src_t = src.reshape(NW, nch, CHUNK)
    dst_t = dst.reshape(NW, nch, CHUNK)

    agg, deg = _sc_segment_sum(x, src_t, dst_t, nch)

    return _tc_finale(
        agg[0], agg[1],
        deg[0].reshape(N_PAD, 1), deg[1].reshape(N_PAD, 1),
        x, W,
        b.reshape(1, D), gamma.reshape(1, D), beta.reshape(1, D),
    )
